# baseline (speedup 1.0000x reference)
import jax
import jax.numpy as jnp
from jax import lax
from jax.experimental import pallas as pl
from jax.experimental.pallas import tpu as pltpu
from jax.experimental.pallas import tpu_sc as plsc

N_NODES = 10000
N_EDGES = 320000
D = 128

NC = 2
NS = 16
NW = NC * NS

E_PER_W = N_EDGES // NW
CHUNK = 80
N_CHUNKS = E_PER_W // CHUNK
S = 5
G = 3
N_GROUPS = N_CHUNKS // S


def _sc_gather(f_hbm, idx_hbm, out_m, out_root, idx_v, *rest):
    buf = rest[0:S]
    sg = rest[S:2 * S]
    swm = rest[2 * S:3 * S]
    swr = rest[3 * S:4 * S]

    wid = lax.axis_index("s") * NC + lax.axis_index("c")
    base = wid * E_PER_W
    pltpu.sync_copy(idx_hbm.at[wid], idx_v)

    def fire_gather(j, s):
        pltpu.make_async_copy(f_hbm.at[idx_v.at[j]], buf[s], sg[s]).start()

    def wait_gather(s):
        pltpu.make_async_copy(f_hbm.at[idx_v.at[0]], buf[s], sg[s]).wait()

    def fire_write(j, s):
        rows = pl.ds(base + j * CHUNK, CHUNK)
        pltpu.make_async_copy(buf[s].at[:, pl.ds(0, D)],
                              out_m.at[rows], swm[s]).start()
        pltpu.make_async_copy(buf[s].at[:, pl.ds(D, D)],
                              out_root.at[rows], swr[s]).start()

    def wait_write(s):
        rows = pl.ds(base, CHUNK)
        pltpu.make_async_copy(buf[s].at[:, pl.ds(0, D)],
                              out_m.at[rows], swm[s]).wait()
        pltpu.make_async_copy(buf[s].at[:, pl.ds(D, D)],
                              out_root.at[rows], swr[s]).wait()

    for j in range(G):
        fire_gather(j, j)
    for k in range(S):
        wait_gather(k)
        fire_write(k, k)
        s3 = (k + G) % S
        if k + G >= S:
            wait_write(s3)
        fire_gather(k + G, s3)

    def body(g, carry):
        for i in range(S):
            k = g * S + i
            wait_gather(i)
            fire_write(k, i)
            s3 = (i + G) % S
            wait_write(s3)
            fire_gather(k + G, s3)
        return carry

    lax.fori_loop(1, N_GROUPS - 1, body, 0)

    for i in range(S):
        k = (N_GROUPS - 1) * S + i
        wait_gather(i)
        fire_write(k, i)
        if k + G < N_CHUNKS:
            s3 = (i + G) % S
            wait_write(s3)
            fire_gather(k + G, s3)
    for i in range(S):
        wait_write(i)


@jax.jit
def kernel(m, root, edge_index):
    fused = jnp.concatenate([m, root], axis=1)
    src = edge_index[0].astype(jnp.int32).reshape(NW, N_CHUNKS, CHUNK)
    mesh = plsc.VectorSubcoreMesh(core_axis_name="c", subcore_axis_name="s")
    out_ty = (jax.ShapeDtypeStruct((N_EDGES, D), jnp.float32),
              jax.ShapeDtypeStruct((N_EDGES, D), jnp.float32))
    f = pl.kernel(
        _sc_gather,
        mesh=mesh,
        out_type=out_ty,
        scratch_types=[
            pltpu.VMEM((N_CHUNKS, CHUNK), jnp.int32),
        ] + [pltpu.VMEM((CHUNK, 2 * D), jnp.float32)] * S
          + [pltpu.SemaphoreType.DMA] * (3 * S),
    )
    return f(fused, src)


# speedup vs baseline: 1.6318x; 1.6318x over previous
import jax
import jax.numpy as jnp
from jax import lax
from jax.experimental import pallas as pl
from jax.experimental.pallas import tpu as pltpu
from jax.experimental.pallas import tpu_sc as plsc

N_NODES = 10000
N_EDGES = 320000
D = 128

NC = 2
NS = 16

E_PER_W = N_EDGES // NS
CHUNK = 80
N_CHUNKS = E_PER_W // CHUNK
S = 4
STAGE_ROWS = 624
STAGE_TAIL = N_NODES - NS * STAGE_ROWS


def _sc_gather(m_hbm, root_hbm, idx_hbm, out_m, out_root, tab, *rest):
    idxb = rest[0:S]
    buf = rest[S:2 * S]
    sidx = rest[2 * S:3 * S]
    sg = rest[3 * S:4 * S]
    sw = rest[4 * S:5 * S]

    cid = lax.axis_index("c")
    sid = lax.axis_index("s")
    base = sid * E_PER_W

    stripe = pl.ds(pl.multiple_of(sid * STAGE_ROWS, 8), STAGE_ROWS)
    tail = pl.ds(NS * STAGE_ROWS, STAGE_TAIL)

    @pl.when(cid == 0)
    def _stage_m():
        pltpu.sync_copy(m_hbm.at[stripe], tab.at[stripe])
        @pl.when(sid == 0)
        def _tail():
            pltpu.sync_copy(m_hbm.at[tail], tab.at[tail])

    @pl.when(cid == 1)
    def _stage_root():
        pltpu.sync_copy(root_hbm.at[stripe], tab.at[stripe])
        @pl.when(sid == 0)
        def _tail():
            pltpu.sync_copy(root_hbm.at[tail], tab.at[tail])

    plsc.subcore_barrier()

    def run_pipe(out):
        def fire_idx(j, s):
            pltpu.make_async_copy(idx_hbm.at[sid, j], idxb[s], sidx[s]).start()

        def wait_idx(s):
            pltpu.make_async_copy(idx_hbm.at[sid, 0], idxb[s], sidx[s]).wait()

        def fire_gather(s):
            pltpu.make_async_copy(tab.at[idxb[s]], buf[s], sg[s]).start()

        def wait_gather(s):
            pltpu.make_async_copy(tab.at[idxb[s]], buf[s], sg[s]).wait()

        def fire_write(j, s):
            dst = out.at[pl.ds(base + j * CHUNK, CHUNK)]
            pltpu.make_async_copy(buf[s], dst, sw[s]).start()

        def wait_write(s):
            dst = out.at[pl.ds(base, CHUNK)]
            pltpu.make_async_copy(buf[s], dst, sw[s]).wait()

        def step(k, s, *, idx_j=None, gather_j=None, write_wait=True):
            wait_gather(s)
            fire_write(k, s)
            if idx_j is not None:
                fire_idx(idx_j, s)
            if gather_j is not None:
                s2 = (s + 2) % S
                wait_idx(s2)
                if write_wait:
                    wait_write(s2)
                fire_gather(s2)

        for s in range(S):
            fire_idx(s, s)
        for s in range(2):
            wait_idx(s)
            fire_gather(s)
        step(0, 0, idx_j=4, gather_j=2, write_wait=False)
        step(1, 1, idx_j=5, gather_j=3, write_wait=False)
        step(2, 2, idx_j=6, gather_j=4)
        step(3, 3, idx_j=7, gather_j=5)

        def body(g, carry):
            for i in range(S):
                k = g * S + i
                step(k, i, idx_j=k + S, gather_j=k + 2)
            return carry

        lax.fori_loop(1, (N_CHUNKS - 4) // S, body, 0)

        step(244, 0, idx_j=248, gather_j=246)
        step(245, 1, idx_j=249, gather_j=247)
        step(246, 2, gather_j=248)
        step(247, 3, gather_j=249)
        step(248, 0)
        step(249, 1)
        for s in (2, 3, 0, 1):
            wait_write(s)

    @pl.when(cid == 0)
    def _produce_m():
        run_pipe(out_m)

    @pl.when(cid == 1)
    def _produce_root():
        run_pipe(out_root)


@jax.jit
def kernel(m, root, edge_index):
    src = edge_index[0].astype(jnp.int32).reshape(NS, N_CHUNKS, CHUNK)
    mesh = plsc.VectorSubcoreMesh(core_axis_name="c", subcore_axis_name="s")
    out_ty = (jax.ShapeDtypeStruct((N_EDGES, D), jnp.float32),
              jax.ShapeDtypeStruct((N_EDGES, D), jnp.float32))
    f = pl.kernel(
        _sc_gather,
        mesh=mesh,
        out_type=out_ty,
        scratch_types=[
            pltpu.VMEM_SHARED((N_NODES, D), jnp.float32),
        ] + [pltpu.VMEM((CHUNK,), jnp.int32)] * S
          + [pltpu.VMEM((CHUNK, D), jnp.float32)] * S
          + [pltpu.SemaphoreType.DMA] * (3 * S),
    )
    return f(m, root, src)
